# baseline (speedup 1.0000x reference)
import functools

import jax
import jax.numpy as jnp
from jax import lax
from jax.experimental import pallas as pl
from jax.experimental.pallas import tpu as pltpu
from jax.experimental.pallas import tpu_sc as plsc

_NUM_CORES = 2
_NUM_SUBCORES = 16
_NUM_WORKERS = _NUM_CORES * _NUM_SUBCORES
_LANES = 16
_CHUNK = 16
_NBUF = 4
_EBUF = 2
_SC_ROWS = 4096
_TC_BLK = 512


@functools.partial(jax.jit, static_argnums=(3, 4))
def _sc_add_pe(emb2d, idx, pe, sc_rows, dim):
    rows_per_w = sc_rows // _NUM_WORKERS
    steps = rows_per_w // _CHUNK
    assert steps >= 2 * _NBUF and (steps - 4) % _NBUF == 0
    mesh = plsc.VectorSubcoreMesh(core_axis_name="c", subcore_axis_name="s")

    scratch = (
        [pltpu.VMEM((rows_per_w,), jnp.int32)]
        + [pltpu.VMEM((_CHUNK, dim), jnp.float32)] * (_NBUF + _EBUF)
        + [pltpu.SemaphoreType.DMA] * (2 * _NBUF + _EBUF)
    )

    @functools.partial(
        pl.kernel,
        mesh=mesh,
        out_type=jax.ShapeDtypeStruct((sc_rows, dim), jnp.float32),
        scratch_types=scratch,
    )
    def k(emb_hbm, idx_hbm, pe_hbm, out_hbm, idx_v, *bufs_and_sems):
        pe_bufs = bufs_and_sems[:_NBUF]
        emb_bufs = bufs_and_sems[_NBUF : _NBUF + _EBUF]
        rest = bufs_and_sems[_NBUF + _EBUF :]
        sem_g = rest[:_NBUF]
        sem_e = rest[_NBUF : _NBUF + _EBUF]
        sem_o = rest[_NBUF + _EBUF :]

        wid = lax.axis_index("s") * _NUM_CORES + lax.axis_index("c")
        wbase = wid * rows_per_w
        pltpu.sync_copy(idx_hbm.at[pl.ds(wbase, rows_per_w)], idx_v)

        def gather_in(kk, s):
            return pltpu.make_async_copy(
                pe_hbm.at[idx_v.at[pl.ds(kk * _CHUNK, _CHUNK)]], pe_bufs[s], sem_g[s]
            )

        def emb_in(kk, s):
            return pltpu.make_async_copy(
                emb_hbm.at[pl.ds(wbase + kk * _CHUNK, _CHUNK)], emb_bufs[s], sem_e[s]
            )

        def out_cp(kk, s):
            return pltpu.make_async_copy(
                pe_bufs[s], out_hbm.at[pl.ds(wbase + kk * _CHUNK, _CHUNK)], sem_o[s]
            )

        def add_chunk(s, es):
            @pl.loop(0, _CHUNK)
            def _(r):
                for c in range(0, dim, _LANES):
                    plsc.addupdate(
                        pe_bufs[s].at[r, pl.ds(c, _LANES)],
                        emb_bufs[es][r, pl.ds(c, _LANES)],
                    )

        def body(kk, s, es, ps, drain, prefetch):
            gather_in(kk, s).wait()
            emb_in(kk, es).wait()
            add_chunk(s, es)
            out_cp(kk, s).start()
            if drain:
                out_cp(kk - 2, ps).wait()
            if prefetch:
                gather_in(kk + 2, ps).start()
                emb_in(kk + 2, es).start()

        for b in range(2):
            gather_in(b, b).start()
            emb_in(b, b % _EBUF).start()

        for kk in (0, 1):
            body(kk, kk, kk % _EBUF, kk + 2, drain=False, prefetch=True)

        @pl.loop(2, steps - 2, step=_NBUF)
        def _(g):
            for b in range(_NBUF):
                body(
                    g + b,
                    (b + 2) % _NBUF,
                    b % _EBUF,
                    b % _NBUF,
                    drain=True,
                    prefetch=True,
                )

        for kk in (steps - 2, steps - 1):
            body(kk, kk % _NBUF, kk % _EBUF, (kk + 2) % _NBUF, drain=True, prefetch=False)

        for kk in (steps - 2, steps - 1):
            out_cp(kk, kk % _NBUF).wait()

    return k(emb2d, idx, pe)


@functools.partial(jax.jit, static_argnums=(3, 4, 5))
def _tc_add_pe(emb1d, pestarts, pe1d, n_rows, dim, sc_rows):
    nblk = (n_rows - sc_rows) // _TC_BLK
    blk0 = sc_rows // _TC_BLK
    blk_elems = _TC_BLK * dim

    def body(pest_ref, emb_ref, pe_hbm, out_ref, pe_buf, sems):
        j = pl.program_id(0)

        def pe_cp(jj, slot):
            return pltpu.make_async_copy(
                pe_hbm.at[pl.ds(pest_ref[jj] * dim, blk_elems)],
                pe_buf.at[slot],
                sems.at[slot],
            )

        @pl.when(j == 0)
        def _():
            pe_cp(0, 0).start()

        @pl.when(j + 1 < nblk)
        def _():
            pe_cp(j + 1, lax.rem(j + 1, 2)).start()

        slot = lax.rem(j, 2)
        pe_cp(j, slot).wait()
        out_ref[...] = emb_ref[...] + pe_buf[slot]

    return pl.pallas_call(
        body,
        grid=(nblk,),
        in_specs=[
            pl.BlockSpec(memory_space=pltpu.SMEM),
            pl.BlockSpec((blk_elems,), lambda j: (j + blk0,)),
            pl.BlockSpec(memory_space=pl.ANY),
        ],
        out_specs=pl.BlockSpec((blk_elems,), lambda j: (j + blk0,)),
        out_shape=jax.ShapeDtypeStruct((n_rows * dim,), jnp.float32),
        scratch_shapes=[
            pltpu.VMEM((2, blk_elems), jnp.float32),
            pltpu.SemaphoreType.DMA((2,)),
        ],
    )(pestarts, emb1d, pe1d)


@functools.partial(jax.jit, static_argnums=(2, 3))
def _stitch(out_tc1d, out_sc1d, sc_elems, n_elems):
    blk = _TC_BLK * 1024
    nblk = sc_elems // blk

    def body(full_ref, sc_ref, out_ref):
        out_ref[...] = sc_ref[...]

    return pl.pallas_call(
        body,
        grid=(nblk,),
        in_specs=[
            pl.BlockSpec(memory_space=pl.ANY),
            pl.BlockSpec((blk,), lambda j: (j,)),
        ],
        out_specs=pl.BlockSpec((blk,), lambda j: (j,)),
        out_shape=jax.ShapeDtypeStruct((n_elems,), jnp.float32),
        input_output_aliases={0: 0},
    )(out_tc1d, out_sc1d)


def kernel(emb, shift, pe):
    bsz, length, dim = emb.shape
    n_rows = bsz * length
    mid_pos = pe.shape[0] // 2
    shift32 = shift.astype(jnp.int32)
    idx = (mid_pos + jnp.arange(length, dtype=jnp.int32))[None, :] - shift32[:, None]
    idx = idx.reshape(n_rows)
    emb2d = emb.reshape(n_rows, dim)

    tc_row0 = _SC_ROWS + jnp.arange((n_rows - _SC_ROWS) // _TC_BLK, dtype=jnp.int32) * _TC_BLK
    tc_b = tc_row0 // length
    pestarts = mid_pos + (tc_row0 - tc_b * length) - shift32[tc_b]

    out_sc = _sc_add_pe(emb2d, idx, pe, _SC_ROWS, dim)
    out_tc = _tc_add_pe(
        emb2d.reshape(n_rows * dim), pestarts, pe.reshape(-1), n_rows, dim, _SC_ROWS
    )
    out = _stitch(out_tc, out_sc.reshape(-1), _SC_ROWS * dim, n_rows * dim)
    return out.reshape(bsz, length, dim)


# speedup vs baseline: 2.7443x; 2.7443x over previous
import functools

import jax
import jax.numpy as jnp
from jax import lax
from jax.experimental import pallas as pl
from jax.experimental.pallas import tpu as pltpu
from jax.experimental.pallas import tpu_sc as plsc

_NUM_CORES = 2
_NUM_SUBCORES = 16
_NUM_WORKERS = _NUM_CORES * _NUM_SUBCORES
_LANES = 16
_CHUNK = 16
_NBUF = 4
_EBUF = 2
_SC_ROWS = 4096
_TC_BLK = 512


@functools.partial(jax.jit, static_argnums=(3, 4))
def _sc_add_pe(emb2d, idx, pe, sc_rows, dim):
    rows_per_w = sc_rows // _NUM_WORKERS
    steps = rows_per_w // _CHUNK
    assert steps >= 2 * _NBUF and (steps - 4) % _NBUF == 0
    mesh = plsc.VectorSubcoreMesh(core_axis_name="c", subcore_axis_name="s")

    scratch = (
        [pltpu.VMEM((rows_per_w,), jnp.int32)]
        + [pltpu.VMEM((_CHUNK, dim), jnp.float32)] * (_NBUF + _EBUF)
        + [pltpu.SemaphoreType.DMA] * (2 * _NBUF + _EBUF)
    )

    @functools.partial(
        pl.kernel,
        mesh=mesh,
        out_type=jax.ShapeDtypeStruct((sc_rows, dim), jnp.float32),
        scratch_types=scratch,
    )
    def k(emb_hbm, idx_hbm, pe_hbm, out_hbm, idx_v, *bufs_and_sems):
        pe_bufs = bufs_and_sems[:_NBUF]
        emb_bufs = bufs_and_sems[_NBUF : _NBUF + _EBUF]
        rest = bufs_and_sems[_NBUF + _EBUF :]
        sem_g = rest[:_NBUF]
        sem_e = rest[_NBUF : _NBUF + _EBUF]
        sem_o = rest[_NBUF + _EBUF :]

        wid = lax.axis_index("s") * _NUM_CORES + lax.axis_index("c")
        wbase = wid * rows_per_w
        pltpu.sync_copy(idx_hbm.at[pl.ds(wbase, rows_per_w)], idx_v)

        def gather_in(kk, s):
            return pltpu.make_async_copy(
                pe_hbm.at[idx_v.at[pl.ds(kk * _CHUNK, _CHUNK)]], pe_bufs[s], sem_g[s]
            )

        def emb_in(kk, s):
            return pltpu.make_async_copy(
                emb_hbm.at[pl.ds(wbase + kk * _CHUNK, _CHUNK)], emb_bufs[s], sem_e[s]
            )

        def out_cp(kk, s):
            return pltpu.make_async_copy(
                pe_bufs[s], out_hbm.at[pl.ds(wbase + kk * _CHUNK, _CHUNK)], sem_o[s]
            )

        def add_chunk(s, es):
            @pl.loop(0, _CHUNK)
            def _(r):
                for c in range(0, dim, _LANES):
                    plsc.addupdate(
                        pe_bufs[s].at[r, pl.ds(c, _LANES)],
                        emb_bufs[es][r, pl.ds(c, _LANES)],
                    )

        def body(kk, s, es, ps, drain, prefetch):
            gather_in(kk, s).wait()
            emb_in(kk, es).wait()
            add_chunk(s, es)
            out_cp(kk, s).start()
            if drain:
                out_cp(kk - 2, ps).wait()
            if prefetch:
                gather_in(kk + 2, ps).start()
                emb_in(kk + 2, es).start()

        for b in range(2):
            gather_in(b, b).start()
            emb_in(b, b % _EBUF).start()

        for kk in (0, 1):
            body(kk, kk, kk % _EBUF, kk + 2, drain=False, prefetch=True)

        @pl.loop(2, steps - 2, step=_NBUF)
        def _(g):
            for b in range(_NBUF):
                body(
                    g + b,
                    (b + 2) % _NBUF,
                    b % _EBUF,
                    b % _NBUF,
                    drain=True,
                    prefetch=True,
                )

        for kk in (steps - 2, steps - 1):
            body(kk, kk % _NBUF, kk % _EBUF, (kk + 2) % _NBUF, drain=True, prefetch=False)

        for kk in (steps - 2, steps - 1):
            out_cp(kk, kk % _NBUF).wait()

    return k(emb2d, idx, pe)


@functools.partial(jax.jit, static_argnums=(3, 4, 5))
def _tc_add_pe(emb2d, pestarts, pe, n_rows, dim, sc_rows):
    nblk = (n_rows - sc_rows) // _TC_BLK
    blk0 = sc_rows // _TC_BLK

    def body(pest_ref, emb_ref, pe_hbm, out_ref, pe_buf, sems):
        j = pl.program_id(0)

        def pe_cp(jj, slot):
            p = pest_ref[jj]
            q8 = pl.multiple_of(p - lax.rem(p, 8), 8)
            return pltpu.make_async_copy(
                pe_hbm.at[pl.ds(q8, _TC_BLK + 8)],
                pe_buf.at[slot],
                sems.at[slot],
            )

        @pl.when(j == 0)
        def _():
            pe_cp(0, 0).start()

        @pl.when(j + 1 < nblk)
        def _():
            pe_cp(j + 1, lax.rem(j + 1, 2)).start()

        slot = lax.rem(j, 2)
        pe_cp(j, slot).wait()
        r = lax.rem(pest_ref[j], 8)
        slab = pltpu.roll(pe_buf[slot], -r, axis=0)
        out_ref[...] = emb_ref[...] + slab[: _TC_BLK, :]

    return pl.pallas_call(
        body,
        grid=(nblk,),
        in_specs=[
            pl.BlockSpec(memory_space=pltpu.SMEM),
            pl.BlockSpec((_TC_BLK, dim), lambda j: (j + blk0, 0)),
            pl.BlockSpec(memory_space=pl.ANY),
        ],
        out_specs=pl.BlockSpec((_TC_BLK, dim), lambda j: (j + blk0, 0)),
        out_shape=jax.ShapeDtypeStruct((n_rows, dim), jnp.float32),
        scratch_shapes=[
            pltpu.VMEM((2, _TC_BLK + 8, dim), jnp.float32),
            pltpu.SemaphoreType.DMA((2,)),
        ],
    )(pestarts, emb2d, pe)


@functools.partial(jax.jit, static_argnums=(2, 3, 4))
def _stitch(out_tc, out_sc, sc_rows, n_rows, dim):
    nblk = sc_rows // _TC_BLK

    def body(full_ref, sc_ref, out_ref):
        out_ref[...] = sc_ref[...]

    return pl.pallas_call(
        body,
        grid=(nblk,),
        in_specs=[
            pl.BlockSpec(memory_space=pl.ANY),
            pl.BlockSpec((_TC_BLK, dim), lambda j: (j, 0)),
        ],
        out_specs=pl.BlockSpec((_TC_BLK, dim), lambda j: (j, 0)),
        out_shape=jax.ShapeDtypeStruct((n_rows, dim), jnp.float32),
        input_output_aliases={0: 0},
    )(out_tc, out_sc)


def kernel(emb, shift, pe):
    bsz, length, dim = emb.shape
    n_rows = bsz * length
    mid_pos = pe.shape[0] // 2
    shift32 = shift.astype(jnp.int32)
    idx = (mid_pos + jnp.arange(length, dtype=jnp.int32))[None, :] - shift32[:, None]
    idx = idx.reshape(n_rows)
    emb2d = emb.reshape(n_rows, dim)

    tc_row0 = _SC_ROWS + jnp.arange((n_rows - _SC_ROWS) // _TC_BLK, dtype=jnp.int32) * _TC_BLK
    tc_b = tc_row0 // length
    pestarts = mid_pos + (tc_row0 - tc_b * length) - shift32[tc_b]

    out_sc = _sc_add_pe(emb2d, idx, pe, _SC_ROWS, dim)
    out_tc = _tc_add_pe(emb2d, pestarts, pe, n_rows, dim, _SC_ROWS)
    out = _stitch(out_tc, out_sc, _SC_ROWS, n_rows, dim)
    return out.reshape(bsz, length, dim)
